# baseline (speedup 1.0000x reference)
import functools

import jax
import jax.numpy as jnp
from jax import lax
from jax.experimental import pallas as pl
from jax.experimental.pallas import tpu as pltpu
from jax.experimental.pallas import tpu_sc as plsc

HIDDEN = 768
EPS = 1e-05
LANES = 16
NSL = HIDDEN // LANES
NC, NS = 2, 16
NW = NC * NS
CHUNK = 64
VROWS = 8
KPR = CHUNK // VROWS


def _shuffle(x, idx):
    dnums = lax.GatherDimensionNumbers(
        offset_dims=(), collapsed_slice_dims=(0,), start_index_map=(0,)
    )
    return lax.gather(
        x,
        idx[:, None],
        dnums,
        (1,),
        mode=lax.GatherScatterMode.PROMISE_IN_BOUNDS,
    )


def _bcast_sum(*vs):
    vs = list(vs)
    lanes = lax.iota(jnp.int32, LANES)
    for sh in (1, 2, 4, 8):
        idx = jnp.bitwise_xor(lanes, jnp.int32(sh))
        vs = [v + _shuffle(v, idx) for v in vs]
    return vs


def _rsqrt_vec(v):
    i = lax.bitcast_convert_type(v, jnp.int32)
    i = jnp.int32(0x5F3759DF) - lax.shift_right_logical(i, 1)
    y = lax.bitcast_convert_type(i, jnp.float32)
    half = v * jnp.float32(0.5)
    for _ in range(2):
        y = y * (jnp.float32(1.5) - half * y * y)
    return y


def _sc_body(n_rows, ids_hbm, table_hbm, w_hbm, out_hbm, idx_v, rows0, rows1, stats_v, w_v, w_v2, gsem0, gsem1, ssem0, ssem1):
    wid = lax.axis_index("s") * NC + lax.axis_index("c")
    rows_per_w = n_rows // NW
    base = wid * rows_per_w
    pltpu.sync_copy(ids_hbm.at[pl.ds(base, rows_per_w)], idx_v)
    pltpu.sync_copy(w_hbm, w_v)
    pltpu.sync_copy(w_hbm, w_v2)

    n_chunks = rows_per_w // CHUNK

    def start_gather(g, buf, sem):
        pltpu.async_copy(table_hbm.at[idx_v.at[pl.ds(g * CHUNK, CHUNK)]], buf, sem)

    def wait_gather(g, buf, sem):
        pltpu.make_async_copy(
            table_hbm.at[idx_v.at[pl.ds(g * CHUNK, CHUNK)]], buf, sem
        ).wait()

    def start_scatter(g, buf, sem):
        pltpu.async_copy(buf, out_hbm.at[pl.ds(base + g * CHUNK, CHUNK)], sem)

    def wait_scatter(g, buf, sem):
        pltpu.make_async_copy(
            buf, out_hbm.at[pl.ds(base + g * CHUNK, CHUNK)], sem
        ).wait()

    def finalize(s, q):
        mean_v = s * jnp.float32(1.0 / HIDDEN)
        var_v = q * jnp.float32(1.0 / HIDDEN) - mean_v * mean_v
        inv_v = _rsqrt_vec(var_v + jnp.float32(EPS))
        return inv_v, mean_v * inv_v

    zero = jnp.zeros((LANES,), jnp.float32)

    HALF = KPR // 2

    def stats_loop(rows_v):
        def pair_body(i, c):
            r = 2 * i
            s0 = q0 = s1 = q1 = zero
            for j in range(NSL):
                sl = pl.ds(j * LANES, LANES)
                x0 = rows_v[r, sl]
                s0 = s0 + x0
                q0 = q0 + x0 * x0
                x1 = rows_v[r + 1, sl]
                s1 = s1 + x1
                q1 = q1 + x1 * x1
            st0, qt0, st1, qt1 = _bcast_sum(s0, q0, s1, q1)
            inv0, minv0 = finalize(st0, qt0)
            inv1, minv1 = finalize(st1, qt1)
            stats_v[4 * i, :] = inv0
            stats_v[4 * i + 1, :] = minv0
            stats_v[4 * i + 2, :] = inv1
            stats_v[4 * i + 3, :] = minv1
            return c

        lax.fori_loop(0, CHUNK // 2, pair_body, 0)

    def norm_loop(rows_v):
        view = rows_v.reshape(KPR, VROWS, HIDDEN)

        def vrow_body(i, c):
            inv = [stats_v[2 * (VROWS * k + i), :] for k in range(KPR)]
            minv = [stats_v[2 * (VROWS * k + i) + 1, :] for k in range(KPR)]
            for j in range(NSL):
                sl = pl.ds(j * LANES, LANES)
                wv = w_v[sl]
                wv2 = w_v2[sl]
                for k in range(KPR):
                    x = view[k, i, sl]
                    view[k, i, sl] = (x * inv[k] - minv[k]) * (wv if k < HALF else wv2)
            return c

        lax.fori_loop(0, VROWS, vrow_body, 0)

    def process(g, buf, other_buf, gsem, other_gsem, ssem, other_ssem):
        wait_gather(g, buf, gsem)
        stats_loop(buf)

        @pl.when(g >= 1)
        def _():
            wait_scatter(g - 1, other_buf, other_ssem)

        @pl.when(g + 1 < n_chunks)
        def _():
            start_gather(g + 1, other_buf, other_gsem)

        norm_loop(buf)
        start_scatter(g, buf, ssem)

    start_gather(0, rows0, gsem0)

    def pair_body(i, carry):
        g0 = 2 * i
        process(g0, rows0, rows1, gsem0, gsem1, ssem0, ssem1)
        process(g0 + 1, rows1, rows0, gsem1, gsem0, ssem1, ssem0)
        return carry

    lax.fori_loop(0, n_chunks // 2, pair_body, 0)
    wait_scatter(n_chunks - 1, rows1, ssem1)


def kernel(input_ids, table, norm_weight):
    b, s = input_ids.shape
    n_rows = b * s
    ids_flat = input_ids.reshape((n_rows,)).astype(jnp.int32)

    mesh = plsc.VectorSubcoreMesh(core_axis_name="c", subcore_axis_name="s")
    rows_per_w = n_rows // NW

    sc_fn = pl.kernel(
        functools.partial(_sc_body, n_rows),
        out_type=jax.ShapeDtypeStruct((n_rows, HIDDEN), jnp.float32),
        mesh=mesh,
        scratch_types=[
            pltpu.VMEM((rows_per_w,), jnp.int32),
            pltpu.VMEM((CHUNK, HIDDEN), jnp.float32),
            pltpu.VMEM((CHUNK, HIDDEN), jnp.float32),
            pltpu.VMEM((2 * CHUNK, LANES), jnp.float32),
            pltpu.VMEM((HIDDEN,), jnp.float32),
            pltpu.VMEM((HIDDEN,), jnp.float32),
            pltpu.SemaphoreType.DMA,
            pltpu.SemaphoreType.DMA,
            pltpu.SemaphoreType.DMA,
            pltpu.SemaphoreType.DMA,
        ],
    )
    out = sc_fn(ids_flat, table, norm_weight)
    return out.reshape((b, s, HIDDEN))


# speedup vs baseline: 1.0058x; 1.0058x over previous
import functools

import jax
import jax.numpy as jnp
from jax import lax
from jax.experimental import pallas as pl
from jax.experimental.pallas import tpu as pltpu
from jax.experimental.pallas import tpu_sc as plsc

HIDDEN = 768
EPS = 1e-05
LANES = 16
NSL = HIDDEN // LANES
NC, NS = 2, 16
NW = NC * NS
CHUNK = 64
VROWS = 8
KPR = CHUNK // VROWS


def _shuffle(x, idx):
    dnums = lax.GatherDimensionNumbers(
        offset_dims=(), collapsed_slice_dims=(0,), start_index_map=(0,)
    )
    return lax.gather(
        x,
        idx[:, None],
        dnums,
        (1,),
        mode=lax.GatherScatterMode.PROMISE_IN_BOUNDS,
    )


def _bcast_sum(*vs):
    vs = list(vs)
    lanes = lax.iota(jnp.int32, LANES)
    for sh in (1, 2, 4, 8):
        idx = jnp.bitwise_xor(lanes, jnp.int32(sh))
        vs = [v + _shuffle(v, idx) for v in vs]
    return vs


def _rsqrt_vec(v):
    i = lax.bitcast_convert_type(v, jnp.int32)
    i = jnp.int32(0x5F3759DF) - lax.shift_right_logical(i, 1)
    y = lax.bitcast_convert_type(i, jnp.float32)
    half = v * jnp.float32(0.5)
    for _ in range(2):
        y = y * (jnp.float32(1.5) - half * y * y)
    return y


def _sc_body(n_rows, ids_hbm, table_hbm, w_hbm, out_hbm, idx_v, rows0, rows1, stats_v, w_v, w_v2, gsem0, gsem1, ssem0, ssem1):
    wid = lax.axis_index("s") * NC + lax.axis_index("c")
    rows_per_w = n_rows // NW
    base = wid * rows_per_w
    pltpu.sync_copy(ids_hbm.at[pl.ds(base, rows_per_w)], idx_v)
    pltpu.sync_copy(w_hbm, w_v)
    pltpu.sync_copy(w_hbm, w_v2)

    n_chunks = rows_per_w // CHUNK

    def start_gather(g, buf, sem):
        pltpu.async_copy(table_hbm.at[idx_v.at[pl.ds(g * CHUNK, CHUNK)]], buf, sem)

    def wait_gather(g, buf, sem):
        pltpu.make_async_copy(
            table_hbm.at[idx_v.at[pl.ds(g * CHUNK, CHUNK)]], buf, sem
        ).wait()

    def start_scatter(g, buf, sem):
        pltpu.async_copy(buf, out_hbm.at[pl.ds(base + g * CHUNK, CHUNK)], sem)

    def wait_scatter(g, buf, sem):
        pltpu.make_async_copy(
            buf, out_hbm.at[pl.ds(base + g * CHUNK, CHUNK)], sem
        ).wait()

    def finalize(s, q):
        mean_v = s * jnp.float32(1.0 / HIDDEN)
        var_v = q * jnp.float32(1.0 / HIDDEN) - mean_v * mean_v
        inv_v = _rsqrt_vec(var_v + jnp.float32(EPS))
        return inv_v, mean_v * inv_v

    zero = jnp.zeros((LANES,), jnp.float32)

    HALF = KPR // 2

    def stats_loop(rows_v):
        def pair_body(i, c):
            r = 2 * i
            s0 = q0 = s1 = q1 = zero
            for j in range(NSL):
                sl = pl.ds(j * LANES, LANES)
                x0 = rows_v[r, sl]
                s0 = s0 + x0
                q0 = q0 + x0 * x0
                x1 = rows_v[r + 1, sl]
                s1 = s1 + x1
                q1 = q1 + x1 * x1
            st0, qt0, st1, qt1 = _bcast_sum(s0, q0, s1, q1)
            inv0, minv0 = finalize(st0, qt0)
            inv1, minv1 = finalize(st1, qt1)
            stats_v[4 * i, :] = inv0
            stats_v[4 * i + 1, :] = minv0
            stats_v[4 * i + 2, :] = inv1
            stats_v[4 * i + 3, :] = minv1
            return c

        lax.fori_loop(0, CHUNK // 2, pair_body, 0)

    def norm_loop(rows_v):
        view = rows_v.reshape(KPR, VROWS, HIDDEN)

        def vrow_body(i, c):
            inv = [stats_v[2 * (VROWS * k + i), :] for k in range(KPR)]
            minv = [stats_v[2 * (VROWS * k + i) + 1, :] for k in range(KPR)]
            def load_xs(j):
                sl = pl.ds(j * LANES, LANES)
                return [view[k, i, sl] for k in range(KPR)]

            xs = load_xs(0)
            for j in range(NSL):
                xs_next = load_xs(j + 1) if j + 1 < NSL else None
                sl = pl.ds(j * LANES, LANES)
                wv = w_v[sl]
                for k in range(KPR):
                    view[k, i, sl] = (xs[k] * inv[k] - minv[k]) * wv
                xs = xs_next
            return c

        lax.fori_loop(0, VROWS, vrow_body, 0)

    def process(g, buf, other_buf, gsem, other_gsem, ssem, other_ssem):
        wait_gather(g, buf, gsem)
        stats_loop(buf)

        @pl.when(g >= 1)
        def _():
            wait_scatter(g - 1, other_buf, other_ssem)

        @pl.when(g + 1 < n_chunks)
        def _():
            start_gather(g + 1, other_buf, other_gsem)

        norm_loop(buf)
        start_scatter(g, buf, ssem)

    start_gather(0, rows0, gsem0)

    def pair_body(i, carry):
        g0 = 2 * i
        process(g0, rows0, rows1, gsem0, gsem1, ssem0, ssem1)
        process(g0 + 1, rows1, rows0, gsem1, gsem0, ssem1, ssem0)
        return carry

    lax.fori_loop(0, n_chunks // 2, pair_body, 0)
    wait_scatter(n_chunks - 1, rows1, ssem1)


def kernel(input_ids, table, norm_weight):
    b, s = input_ids.shape
    n_rows = b * s
    ids_flat = input_ids.reshape((n_rows,)).astype(jnp.int32)

    mesh = plsc.VectorSubcoreMesh(core_axis_name="c", subcore_axis_name="s")
    rows_per_w = n_rows // NW

    sc_fn = pl.kernel(
        functools.partial(_sc_body, n_rows),
        out_type=jax.ShapeDtypeStruct((n_rows, HIDDEN), jnp.float32),
        mesh=mesh,
        scratch_types=[
            pltpu.VMEM((rows_per_w,), jnp.int32),
            pltpu.VMEM((CHUNK, HIDDEN), jnp.float32),
            pltpu.VMEM((CHUNK, HIDDEN), jnp.float32),
            pltpu.VMEM((2 * CHUNK, LANES), jnp.float32),
            pltpu.VMEM((HIDDEN,), jnp.float32),
            pltpu.VMEM((HIDDEN,), jnp.float32),
            pltpu.SemaphoreType.DMA,
            pltpu.SemaphoreType.DMA,
            pltpu.SemaphoreType.DMA,
            pltpu.SemaphoreType.DMA,
        ],
    )
    out = sc_fn(ids_flat, table, norm_weight)
    return out.reshape((b, s, HIDDEN))


# speedup vs baseline: 1.0064x; 1.0006x over previous
import functools

import jax
import jax.numpy as jnp
from jax import lax
from jax.experimental import pallas as pl
from jax.experimental.pallas import tpu as pltpu
from jax.experimental.pallas import tpu_sc as plsc

HIDDEN = 768
EPS = 1e-05
LANES = 16
NSL = HIDDEN // LANES
NC, NS = 2, 16
NW = NC * NS
CHUNK = 64
VROWS = 8
KPR = CHUNK // VROWS


def _shuffle(x, idx):
    dnums = lax.GatherDimensionNumbers(
        offset_dims=(), collapsed_slice_dims=(0,), start_index_map=(0,)
    )
    return lax.gather(
        x,
        idx[:, None],
        dnums,
        (1,),
        mode=lax.GatherScatterMode.PROMISE_IN_BOUNDS,
    )


def _bcast_sum(*vs):
    vs = list(vs)
    lanes = lax.iota(jnp.int32, LANES)
    for sh in (1, 2, 4, 8):
        idx = jnp.bitwise_xor(lanes, jnp.int32(sh))
        vs = [v + _shuffle(v, idx) for v in vs]
    return vs


def _rsqrt_vec(v):
    i = lax.bitcast_convert_type(v, jnp.int32)
    i = jnp.int32(0x5F3759DF) - lax.shift_right_logical(i, 1)
    y = lax.bitcast_convert_type(i, jnp.float32)
    half = v * jnp.float32(0.5)
    for _ in range(2):
        y = y * (jnp.float32(1.5) - half * y * y)
    return y


def _sc_body(n_rows, ids_hbm, table_hbm, w_hbm, out_hbm, idx_v, rows0, rows1, stats_v, w_v, gsem0, gsem1, ssem0, ssem1):
    wid = lax.axis_index("s") * NC + lax.axis_index("c")
    rows_per_w = n_rows // NW
    base = wid * rows_per_w
    pltpu.sync_copy(ids_hbm.at[pl.ds(base, rows_per_w)], idx_v)
    pltpu.sync_copy(w_hbm, w_v)

    n_chunks = rows_per_w // CHUNK

    def start_gather(g, buf, sem):
        pltpu.async_copy(table_hbm.at[idx_v.at[pl.ds(g * CHUNK, CHUNK)]], buf, sem)

    def wait_gather(g, buf, sem):
        pltpu.make_async_copy(
            table_hbm.at[idx_v.at[pl.ds(g * CHUNK, CHUNK)]], buf, sem
        ).wait()

    def start_scatter(g, buf, sem):
        pltpu.async_copy(buf, out_hbm.at[pl.ds(base + g * CHUNK, CHUNK)], sem)

    def wait_scatter(g, buf, sem):
        pltpu.make_async_copy(
            buf, out_hbm.at[pl.ds(base + g * CHUNK, CHUNK)], sem
        ).wait()

    def finalize(s, q):
        mean_v = s * jnp.float32(1.0 / HIDDEN)
        var_v = q * jnp.float32(1.0 / HIDDEN) - mean_v * mean_v
        inv_v = _rsqrt_vec(var_v + jnp.float32(EPS))
        return inv_v, mean_v * inv_v

    zero = jnp.zeros((LANES,), jnp.float32)

    def stats_loop(rows_v):
        def pair_body(i, c):
            r = 2 * i
            s0 = q0 = s1 = q1 = zero
            for j in range(NSL):
                sl = pl.ds(j * LANES, LANES)
                x0 = rows_v[r, sl]
                s0 = s0 + x0
                q0 = q0 + x0 * x0
                x1 = rows_v[r + 1, sl]
                s1 = s1 + x1
                q1 = q1 + x1 * x1
            st0, qt0, st1, qt1 = _bcast_sum(s0, q0, s1, q1)
            inv0, minv0 = finalize(st0, qt0)
            inv1, minv1 = finalize(st1, qt1)
            stats_v[4 * i, :] = inv0
            stats_v[4 * i + 1, :] = minv0
            stats_v[4 * i + 2, :] = inv1
            stats_v[4 * i + 3, :] = minv1
            return c

        lax.fori_loop(0, CHUNK // 2, pair_body, 0)

    def norm_loop(rows_v):
        view = rows_v.reshape(KPR, VROWS, HIDDEN)

        def vrow_body(i, c):
            inv = [stats_v[2 * (VROWS * k + i), :] for k in range(KPR)]
            minv = [stats_v[2 * (VROWS * k + i) + 1, :] for k in range(KPR)]
            for j in range(NSL):
                sl = pl.ds(j * LANES, LANES)
                wv = w_v[sl]
                for k in range(KPR):
                    x = view[k, i, sl]
                    view[k, i, sl] = (x * inv[k] - minv[k]) * wv
            return c

        lax.fori_loop(0, VROWS, vrow_body, 0)

    def process(g, buf, other_buf, gsem, other_gsem, ssem, other_ssem):
        wait_gather(g, buf, gsem)
        stats_loop(buf)

        @pl.when(g >= 1)
        def _():
            wait_scatter(g - 1, other_buf, other_ssem)

        @pl.when(g + 1 < n_chunks)
        def _():
            start_gather(g + 1, other_buf, other_gsem)

        norm_loop(buf)
        start_scatter(g, buf, ssem)

    start_gather(0, rows0, gsem0)

    def pair_body(i, carry):
        g0 = 2 * i
        process(g0, rows0, rows1, gsem0, gsem1, ssem0, ssem1)
        process(g0 + 1, rows1, rows0, gsem1, gsem0, ssem1, ssem0)
        return carry

    lax.fori_loop(0, n_chunks // 2, pair_body, 0)
    wait_scatter(n_chunks - 1, rows1, ssem1)


def kernel(input_ids, table, norm_weight):
    b, s = input_ids.shape
    n_rows = b * s
    ids_flat = input_ids.reshape((n_rows,)).astype(jnp.int32)

    mesh = plsc.VectorSubcoreMesh(core_axis_name="c", subcore_axis_name="s")
    rows_per_w = n_rows // NW

    sc_fn = pl.kernel(
        functools.partial(_sc_body, n_rows),
        out_type=jax.ShapeDtypeStruct((n_rows, HIDDEN), jnp.float32),
        mesh=mesh,
        scratch_types=[
            pltpu.VMEM((rows_per_w,), jnp.int32),
            pltpu.VMEM((CHUNK, HIDDEN), jnp.float32),
            pltpu.VMEM((CHUNK, HIDDEN), jnp.float32),
            pltpu.VMEM((2 * CHUNK, LANES), jnp.float32),
            pltpu.VMEM((HIDDEN,), jnp.float32),
            pltpu.SemaphoreType.DMA,
            pltpu.SemaphoreType.DMA,
            pltpu.SemaphoreType.DMA,
            pltpu.SemaphoreType.DMA,
        ],
    )
    out = sc_fn(ids_flat, table, norm_weight)
    return out.reshape((b, s, HIDDEN))


# speedup vs baseline: 1.0780x; 1.0711x over previous
import functools

import jax
import jax.numpy as jnp
from jax import lax
from jax.experimental import pallas as pl
from jax.experimental.pallas import tpu as pltpu
from jax.experimental.pallas import tpu_sc as plsc

HIDDEN = 768
EPS = 1e-05
LANES = 16
NSL = HIDDEN // LANES
NC, NS = 2, 16
NW = NC * NS
CHUNK = 64
VROWS = 8
KPR = CHUNK // VROWS


def _shuffle(x, idx):
    dnums = lax.GatherDimensionNumbers(
        offset_dims=(), collapsed_slice_dims=(0,), start_index_map=(0,)
    )
    return lax.gather(
        x,
        idx[:, None],
        dnums,
        (1,),
        mode=lax.GatherScatterMode.PROMISE_IN_BOUNDS,
    )


def _bcast_sum(*vs):
    vs = list(vs)
    lanes = lax.iota(jnp.int32, LANES)
    for sh in (1, 2, 4, 8):
        idx = jnp.bitwise_xor(lanes, jnp.int32(sh))
        vs = [v + _shuffle(v, idx) for v in vs]
    return vs


def _rsqrt_vec(v):
    i = lax.bitcast_convert_type(v, jnp.int32)
    i = jnp.int32(0x5F3759DF) - lax.shift_right_logical(i, 1)
    y = lax.bitcast_convert_type(i, jnp.float32)
    half = v * jnp.float32(0.5)
    for _ in range(2):
        y = y * (jnp.float32(1.5) - half * y * y)
    return y


def _sc_body(n_rows, ids_hbm, table_hbm, w_hbm, out_hbm, idx_v, rows0, rows1, stats_v, w_v, gsem0, gsem1, ssem0, ssem1):
    wid = lax.axis_index("s") * NC + lax.axis_index("c")
    rows_per_w = n_rows // NW
    base = wid * rows_per_w
    pltpu.sync_copy(ids_hbm.at[pl.ds(base, rows_per_w)], idx_v)
    pltpu.sync_copy(w_hbm, w_v)

    n_chunks = rows_per_w // CHUNK

    def start_gather(g, buf, sem):
        pltpu.async_copy(table_hbm.at[idx_v.at[pl.ds(g * CHUNK, CHUNK)]], buf, sem)

    def wait_gather(g, buf, sem):
        pltpu.make_async_copy(
            table_hbm.at[idx_v.at[pl.ds(g * CHUNK, CHUNK)]], buf, sem
        ).wait()

    def start_scatter(g, buf, sem):
        pltpu.async_copy(buf, out_hbm.at[pl.ds(base + g * CHUNK, CHUNK)], sem)

    def wait_scatter(g, buf, sem):
        pltpu.make_async_copy(
            buf, out_hbm.at[pl.ds(base + g * CHUNK, CHUNK)], sem
        ).wait()

    def finalize(s, q):
        mean_v = s * jnp.float32(1.0 / HIDDEN)
        var_v = q * jnp.float32(1.0 / HIDDEN) - mean_v * mean_v
        inv_v = _rsqrt_vec(var_v + jnp.float32(EPS))
        return inv_v, mean_v * inv_v

    zero = jnp.zeros((LANES,), jnp.float32)

    def stats_loop(rows_v):
        def quad_body(i, c):
            r = 4 * i
            s = [zero] * 4
            q = [zero] * 4
            for j in range(NSL):
                sl = pl.ds(j * LANES, LANES)
                for n in range(4):
                    x = rows_v[r + n, sl]
                    s[n] = s[n] + x
                    q[n] = q[n] + x * x
            tot = _bcast_sum(*s, *q)
            for n in range(4):
                inv, minv = finalize(tot[n], tot[4 + n])
                stats_v[8 * i + 2 * n, :] = inv
                stats_v[8 * i + 2 * n + 1, :] = minv
            return c

        lax.fori_loop(0, CHUNK // 4, quad_body, 0)

    def norm_loop(rows_v):
        view = rows_v.reshape(KPR, VROWS, HIDDEN)

        def vrow_body(i, c):
            inv = [stats_v[2 * (VROWS * k + i), :] for k in range(KPR)]
            minv = [stats_v[2 * (VROWS * k + i) + 1, :] for k in range(KPR)]
            for j in range(NSL):
                sl = pl.ds(j * LANES, LANES)
                wv = w_v[sl]
                for k in range(KPR):
                    x = view[k, i, sl]
                    view[k, i, sl] = (x * inv[k] - minv[k]) * wv
            return c

        lax.fori_loop(0, VROWS, vrow_body, 0)

    def process(g, buf, other_buf, gsem, other_gsem, ssem, other_ssem):
        wait_gather(g, buf, gsem)
        stats_loop(buf)

        @pl.when(g >= 1)
        def _():
            wait_scatter(g - 1, other_buf, other_ssem)

        @pl.when(g + 1 < n_chunks)
        def _():
            start_gather(g + 1, other_buf, other_gsem)

        norm_loop(buf)
        start_scatter(g, buf, ssem)

    start_gather(0, rows0, gsem0)

    def pair_body(i, carry):
        g0 = 2 * i
        process(g0, rows0, rows1, gsem0, gsem1, ssem0, ssem1)
        process(g0 + 1, rows1, rows0, gsem1, gsem0, ssem1, ssem0)
        return carry

    lax.fori_loop(0, n_chunks // 2, pair_body, 0)
    wait_scatter(n_chunks - 1, rows1, ssem1)


def kernel(input_ids, table, norm_weight):
    b, s = input_ids.shape
    n_rows = b * s
    ids_flat = input_ids.reshape((n_rows,)).astype(jnp.int32)

    mesh = plsc.VectorSubcoreMesh(core_axis_name="c", subcore_axis_name="s")
    rows_per_w = n_rows // NW

    sc_fn = pl.kernel(
        functools.partial(_sc_body, n_rows),
        out_type=jax.ShapeDtypeStruct((n_rows, HIDDEN), jnp.float32),
        mesh=mesh,
        scratch_types=[
            pltpu.VMEM((rows_per_w,), jnp.int32),
            pltpu.VMEM((CHUNK, HIDDEN), jnp.float32),
            pltpu.VMEM((CHUNK, HIDDEN), jnp.float32),
            pltpu.VMEM((2 * CHUNK, LANES), jnp.float32),
            pltpu.VMEM((HIDDEN,), jnp.float32),
            pltpu.SemaphoreType.DMA,
            pltpu.SemaphoreType.DMA,
            pltpu.SemaphoreType.DMA,
            pltpu.SemaphoreType.DMA,
        ],
    )
    out = sc_fn(ids_flat, table, norm_weight)
    return out.reshape((b, s, HIDDEN))


# speedup vs baseline: 1.0785x; 1.0005x over previous
import functools

import jax
import jax.numpy as jnp
from jax import lax
from jax.experimental import pallas as pl
from jax.experimental.pallas import tpu as pltpu
from jax.experimental.pallas import tpu_sc as plsc

HIDDEN = 768
EPS = 1e-05
LANES = 16
NSL = HIDDEN // LANES
NC, NS = 2, 16
NW = NC * NS
CHUNK = 64
VROWS = 8
KPR = CHUNK // VROWS


def _shuffle(x, idx):
    dnums = lax.GatherDimensionNumbers(
        offset_dims=(), collapsed_slice_dims=(0,), start_index_map=(0,)
    )
    return lax.gather(
        x,
        idx[:, None],
        dnums,
        (1,),
        mode=lax.GatherScatterMode.PROMISE_IN_BOUNDS,
    )


def _bcast_sum(*vs):
    vs = list(vs)
    lanes = lax.iota(jnp.int32, LANES)
    for sh in (1, 2, 4, 8):
        idx = jnp.bitwise_xor(lanes, jnp.int32(sh))
        vs = [v + _shuffle(v, idx) for v in vs]
    return vs


def _rsqrt_vec(v):
    i = lax.bitcast_convert_type(v, jnp.int32)
    i = jnp.int32(0x5F3759DF) - lax.shift_right_logical(i, 1)
    y = lax.bitcast_convert_type(i, jnp.float32)
    half = v * jnp.float32(0.5)
    for _ in range(2):
        y = y * (jnp.float32(1.5) - half * y * y)
    return y


def _sc_body(n_rows, ids_hbm, table_hbm, w_hbm, out_hbm, idx_v, rows0, rows1, stats_v, w_v, gsem0, gsem1, ssem0, ssem1):
    wid = lax.axis_index("s") * NC + lax.axis_index("c")
    rows_per_w = n_rows // NW
    base = wid * rows_per_w
    pltpu.sync_copy(ids_hbm.at[pl.ds(base, rows_per_w)], idx_v)
    pltpu.sync_copy(w_hbm, w_v)

    n_chunks = rows_per_w // CHUNK

    def start_gather(g, buf, sem):
        pltpu.async_copy(table_hbm.at[idx_v.at[pl.ds(g * CHUNK, CHUNK)]], buf, sem)

    def wait_gather(g, buf, sem):
        pltpu.make_async_copy(
            table_hbm.at[idx_v.at[pl.ds(g * CHUNK, CHUNK)]], buf, sem
        ).wait()

    def start_scatter(g, buf, sem):
        pltpu.async_copy(buf, out_hbm.at[pl.ds(base + g * CHUNK, CHUNK)], sem)

    def wait_scatter(g, buf, sem):
        pltpu.make_async_copy(
            buf, out_hbm.at[pl.ds(base + g * CHUNK, CHUNK)], sem
        ).wait()

    def finalize(s, q):
        mean_v = s * jnp.float32(1.0 / HIDDEN)
        var_v = q * jnp.float32(1.0 / HIDDEN) - mean_v * mean_v
        inv_v = _rsqrt_vec(var_v + jnp.float32(EPS))
        return inv_v, mean_v * inv_v

    zero = jnp.zeros((LANES,), jnp.float32)

    def stats_loop(rows_v):
        SG = 8

        def group_body(i, c):
            r = SG * i
            s = [zero] * SG
            q = [zero] * SG
            for j in range(NSL):
                sl = pl.ds(j * LANES, LANES)
                for n in range(SG):
                    x = rows_v[r + n, sl]
                    s[n] = s[n] + x
                    q[n] = q[n] + x * x
            tot = _bcast_sum(*s, *q)
            for n in range(SG):
                inv, minv = finalize(tot[n], tot[SG + n])
                stats_v[2 * SG * i + 2 * n, :] = inv
                stats_v[2 * SG * i + 2 * n + 1, :] = minv
            return c

        lax.fori_loop(0, CHUNK // SG, group_body, 0)

    def norm_loop(rows_v):
        view = rows_v.reshape(KPR, VROWS, HIDDEN)

        def vrow_body(i, c):
            inv = [stats_v[2 * (VROWS * k + i), :] for k in range(KPR)]
            minv = [stats_v[2 * (VROWS * k + i) + 1, :] for k in range(KPR)]
            for j in range(NSL):
                sl = pl.ds(j * LANES, LANES)
                wv = w_v[sl]
                for k in range(KPR):
                    x = view[k, i, sl]
                    view[k, i, sl] = (x * inv[k] - minv[k]) * wv
            return c

        lax.fori_loop(0, VROWS, vrow_body, 0)

    def process(g, buf, other_buf, gsem, other_gsem, ssem, other_ssem):
        wait_gather(g, buf, gsem)
        stats_loop(buf)

        @pl.when(g >= 1)
        def _():
            wait_scatter(g - 1, other_buf, other_ssem)

        @pl.when(g + 1 < n_chunks)
        def _():
            start_gather(g + 1, other_buf, other_gsem)

        norm_loop(buf)
        start_scatter(g, buf, ssem)

    start_gather(0, rows0, gsem0)

    def pair_body(i, carry):
        g0 = 2 * i
        process(g0, rows0, rows1, gsem0, gsem1, ssem0, ssem1)
        process(g0 + 1, rows1, rows0, gsem1, gsem0, ssem1, ssem0)
        return carry

    lax.fori_loop(0, n_chunks // 2, pair_body, 0)
    wait_scatter(n_chunks - 1, rows1, ssem1)


def kernel(input_ids, table, norm_weight):
    b, s = input_ids.shape
    n_rows = b * s
    ids_flat = input_ids.reshape((n_rows,)).astype(jnp.int32)

    mesh = plsc.VectorSubcoreMesh(core_axis_name="c", subcore_axis_name="s")
    rows_per_w = n_rows // NW

    sc_fn = pl.kernel(
        functools.partial(_sc_body, n_rows),
        out_type=jax.ShapeDtypeStruct((n_rows, HIDDEN), jnp.float32),
        mesh=mesh,
        scratch_types=[
            pltpu.VMEM((rows_per_w,), jnp.int32),
            pltpu.VMEM((CHUNK, HIDDEN), jnp.float32),
            pltpu.VMEM((CHUNK, HIDDEN), jnp.float32),
            pltpu.VMEM((2 * CHUNK, LANES), jnp.float32),
            pltpu.VMEM((HIDDEN,), jnp.float32),
            pltpu.SemaphoreType.DMA,
            pltpu.SemaphoreType.DMA,
            pltpu.SemaphoreType.DMA,
            pltpu.SemaphoreType.DMA,
        ],
    )
    out = sc_fn(ids_flat, table, norm_weight)
    return out.reshape((b, s, HIDDEN))
